# Initial kernel scaffold; baseline (speedup 1.0000x reference)
import jax
import jax.numpy as jnp
from jax.experimental import pallas as pl


def kernel(xyz, features, edges, W1, b1, g1, be1, W2, b2, g2, be2):
    raise NotImplementedError("write your pallas kernel here")


# speedup vs baseline: 4.7520x; 4.7520x over previous
import functools

import jax
import jax.numpy as jnp
from jax import lax
from jax.experimental import pallas as pl
from jax.experimental.pallas import tpu as pltpu
from jax.experimental.pallas import tpu_sc as plsc

N = 10000
C = 128
E = 320000
EPS = 1e-5

NC = 2
NS = 16
L = 16
NW = NC * NS
NPAD = 10240
R = NPAD // NW
CH = 4000
NG = CH // L
GB = 64
NCHUNK = E // CH
KC = C // L

_NEG = float("-inf")


def _tc1_body(ivf_ref, xyz_ref, w1a_ref, w1b_ref, b1_ref, a_ref, q_ref):
    xyz = xyz_ref[...]
    w1b = w1b_ref[...]
    q = (xyz[:, 0:1] * w1b[0:1, :] + xyz[:, 1:2] * w1b[1:2, :]
         + xyz[:, 2:3] * w1b[2:3, :])
    a = jnp.dot(ivf_ref[...], w1a_ref[...],
                preferred_element_type=jnp.float32) + q + b1_ref[...]
    q_ref[...] = q
    a_ref[...] = a


_TC1_BLK = 2560


def _tc1(ivf_p, xyz_p, w1a_t, w1b_t, b1_2d):
    nblk = NPAD // _TC1_BLK
    return pl.pallas_call(
        _tc1_body,
        grid=(nblk,),
        in_specs=[
            pl.BlockSpec((_TC1_BLK, C), lambda i: (i, 0)),
            pl.BlockSpec((_TC1_BLK, 4), lambda i: (i, 0)),
            pl.BlockSpec((C, C), lambda i: (0, 0)),
            pl.BlockSpec((4, C), lambda i: (0, 0)),
            pl.BlockSpec((1, C), lambda i: (0, 0)),
        ],
        out_specs=[
            pl.BlockSpec((_TC1_BLK, C), lambda i: (i, 0)),
            pl.BlockSpec((_TC1_BLK, C), lambda i: (i, 0)),
        ],
        out_shape=[
            jax.ShapeDtypeStruct((NPAD, C), jnp.float32),
            jax.ShapeDtypeStruct((NPAD, C), jnp.float32),
        ],
    )(ivf_p, jnp.pad(xyz_p, ((0, 0), (0, 1))), w1a_t,
      jnp.pad(w1b_t, ((0, 1), (0, 0))), b1_2d)


def _sc_body(a_hbm, q_hbm, src_hbm, dst_hbm,
             m_hbm, st_hbm,
             q_blk, m_blk, src_st, dst_st, lst_s, lst_d, arows, stat_st):
    cid = lax.axis_index("c")
    sid = lax.axis_index("s")
    wid = sid * NC + cid
    base = wid * R

    neg = jnp.full((L,), _NEG, jnp.float32)
    zero = jnp.zeros((L,), jnp.float32)

    @pl.loop(0, R)
    def _(r):
        for k in range(KC):
            m_blk[r, pl.ds(k * L, L)] = neg

    for k in range(KC):
        stat_st[0, pl.ds(k * L, L)] = zero
        stat_st[1, pl.ds(k * L, L)] = zero

    zeroi = jnp.zeros((L,), jnp.int32)

    @pl.loop(0, (CH + L) // L)
    def _(i):
        lst_s[pl.ds(i * L, L)] = zeroi
        lst_d[pl.ds(i * L, L)] = zeroi

    pltpu.sync_copy(q_hbm.at[pl.ds(base, R)], q_blk)

    @pl.loop(0, NCHUNK)
    def _(ci):
        pltpu.sync_copy(src_hbm.at[pl.ds(ci * CH, CH)], src_st)
        pltpu.sync_copy(dst_hbm.at[pl.ds(ci * CH, CH)], dst_st)

        def fbody(g, pos):
            vd = dst_st[pl.ds(g * L, L)]
            vs = src_st[pl.ds(g * L, L)]
            dl = vd - base
            mask = (dl >= 0) & (dl < R)
            plsc.store_compressed(lst_s.at[pl.ds(pos, L)], vs, mask=mask)
            plsc.store_compressed(lst_d.at[pl.ds(pos, L)], dl, mask=mask)
            cnt = jnp.sum(jnp.where(mask, 1, 0).astype(jnp.int32), axis=0)
            return pos + cnt

        cnt = lax.fori_loop(0, NG, fbody, jnp.int32(0))

        nb = (cnt + (GB - 1)) // GB

        def bbody(b, stats):
            pltpu.sync_copy(a_hbm.at[lst_s.at[pl.ds(b * GB, GB)]], arows)

            def gbody(g, stats):
                erow = g * L
                vd = lst_d[pl.ds(b * GB + erow, L)]
                new = list(stats)
                for k in range(L):
                    d = vd[k]
                    valid = (b * GB + erow + k) < cnt
                    for kc in range(KC):
                        sl = pl.ds(kc * L, L)
                        a = arows[erow + k, sl]
                        a_eff = jnp.where(valid, a, _NEG)
                        m_blk[d, sl] = jnp.maximum(m_blk[d, sl], a_eff)
                        h = jnp.maximum(a_eff - q_blk[d, sl], 0.0)
                        new[kc] = new[kc] + h
                        new[KC + kc] = new[KC + kc] + h * h
                return tuple(new)

            return lax.fori_loop(0, GB // L, gbody, stats)

        stats0 = tuple(jnp.zeros((L,), jnp.float32) for _ in range(2 * KC))
        stats = lax.fori_loop(0, nb, bbody, stats0)

        for k in range(KC):
            sl = pl.ds(k * L, L)
            stat_st[0, sl] = stat_st[0, sl] + stats[k]
            stat_st[1, sl] = stat_st[1, sl] + stats[KC + k]

    pltpu.sync_copy(m_blk, m_hbm.at[pl.ds(base, R)])
    pltpu.sync_copy(stat_st, st_hbm.at[wid])


def _sc_call(a, q, src, dst):
    mesh = plsc.VectorSubcoreMesh(core_axis_name="c", subcore_axis_name="s")
    cp = pltpu.CompilerParams()
    if "needs_layout_passes" in pltpu.CompilerParams.__dataclass_fields__:
        import dataclasses
        cp = dataclasses.replace(cp, needs_layout_passes=False)
    kern = pl.kernel(
        _sc_body,
        out_type=[
            jax.ShapeDtypeStruct((NPAD, C), jnp.float32),
            jax.ShapeDtypeStruct((NW, 2, C), jnp.float32),
        ],
        mesh=mesh,
        scratch_types=[
            pltpu.VMEM((R, C), jnp.float32),
            pltpu.VMEM((R, C), jnp.float32),
            pltpu.VMEM((CH,), jnp.int32),
            pltpu.VMEM((CH,), jnp.int32),
            pltpu.VMEM((CH + L,), jnp.int32),
            pltpu.VMEM((CH + L,), jnp.int32),
            pltpu.VMEM((GB, C), jnp.float32),
            pltpu.VMEM((2, C), jnp.float32),
        ],
        compiler_params=cp,
    )
    return kern(a, q, src, dst)


def _tc2_body(m_ref, q_ref, st_ref, ivf_ref, w2t_ref, b2_ref,
              g1_ref, be1_ref, g2_ref, be2_ref, o_ref):
    st = st_ref[...]
    s = jnp.sum(st[:, 0, :], axis=0, keepdims=True)
    ss = jnp.sum(st[:, 1, :], axis=0, keepdims=True)
    mu1 = s / E
    var1 = ss / E - mu1 * mu1
    inv1 = g1_ref[...] * lax.rsqrt(var1 + EPS)

    m = m_ref[...]
    hseg = jnp.maximum(m - q_ref[...], 0.0)
    agg = jnp.where(m == _NEG, 0.0, (hseg - mu1) * inv1 + be1_ref[...])

    u = jnp.dot(agg, w2t_ref[...], preferred_element_type=jnp.float32)
    u = jnp.maximum(u + b2_ref[...], 0.0)

    rows = lax.broadcasted_iota(jnp.int32, (NPAD, 1), 0)
    valid = rows < N
    uv = jnp.where(valid, u, 0.0)
    mu2 = jnp.sum(uv, axis=0, keepdims=True) / N
    dev = jnp.where(valid, u - mu2, 0.0)
    var2 = jnp.sum(dev * dev, axis=0, keepdims=True) / N
    inv2 = g2_ref[...] * lax.rsqrt(var2 + EPS)
    o_ref[...] = (u - mu2) * inv2 + be2_ref[...] + ivf_ref[...]


def _tc2(m, q, st, ivf_p, w2_t, b2_2d, g1_2d, be1_2d, g2_2d, be2_2d):
    return pl.pallas_call(
        _tc2_body,
        out_shape=jax.ShapeDtypeStruct((NPAD, C), jnp.float32),
    )(m, q, st, ivf_p, w2_t, b2_2d, g1_2d, be1_2d, g2_2d, be2_2d)


@jax.jit
def kernel(xyz, features, edges, W1, b1, g1, be1, W2, b2, g2, be2):
    f = jnp.float32
    ivf = features[0].astype(f).T
    ivf_p = jnp.pad(ivf, ((0, NPAD - N), (0, 0)))
    xyz_p = jnp.pad(xyz[0].astype(f), ((0, NPAD - N), (0, 0)))
    src = edges[0, 0].astype(jnp.int32)
    dst = edges[0, 1].astype(jnp.int32)

    w1a_t = W1[:, :C].astype(f).T
    w1b_t = W1[:, C:].astype(f).T

    a, q = _tc1(ivf_p, xyz_p, w1a_t, w1b_t, b1.astype(f)[None])
    m, st = _sc_call(a, q, src, dst)
    res = _tc2(m, q, st, ivf_p, W2.astype(f).T, b2.astype(f)[None],
               g1.astype(f)[None], be1.astype(f)[None],
               g2.astype(f)[None], be2.astype(f)[None])
    return res[:N].T[None]


# speedup vs baseline: 6.9306x; 1.4585x over previous
import functools

import jax
import jax.numpy as jnp
from jax import lax
from jax.experimental import pallas as pl
from jax.experimental.pallas import tpu as pltpu
from jax.experimental.pallas import tpu_sc as plsc

N = 10000
C = 128
E = 320000
EPS = 1e-5

NC = 2
NS = 16
L = 16
NW = NC * NS
NPAD = 10240
R = NPAD // NW
CH = 2560
NG = CH // L
GB = 96
LCAP = ((CH + GB - 1) // GB) * GB
NCHUNK = E // CH
KC = C // L

_NEG = float("-inf")


def _tc1_body(ivf_ref, xyz_ref, w1a_ref, w1b_ref, b1_ref, a_ref, q_ref):
    xyz = xyz_ref[...]
    w1b = w1b_ref[...]
    q = (xyz[:, 0:1] * w1b[0:1, :] + xyz[:, 1:2] * w1b[1:2, :]
         + xyz[:, 2:3] * w1b[2:3, :])
    a = jnp.dot(ivf_ref[...], w1a_ref[...],
                preferred_element_type=jnp.float32) + q + b1_ref[...]
    q_ref[...] = q
    a_ref[...] = a


_TC1_BLK = 2560


def _tc1(ivf_p, xyz_p, w1a_t, w1b_t, b1_2d):
    nblk = NPAD // _TC1_BLK
    return pl.pallas_call(
        _tc1_body,
        grid=(nblk,),
        in_specs=[
            pl.BlockSpec((_TC1_BLK, C), lambda i: (i, 0)),
            pl.BlockSpec((_TC1_BLK, 4), lambda i: (i, 0)),
            pl.BlockSpec((C, C), lambda i: (0, 0)),
            pl.BlockSpec((4, C), lambda i: (0, 0)),
            pl.BlockSpec((1, C), lambda i: (0, 0)),
        ],
        out_specs=[
            pl.BlockSpec((_TC1_BLK, C), lambda i: (i, 0)),
            pl.BlockSpec((_TC1_BLK, C), lambda i: (i, 0)),
        ],
        out_shape=[
            jax.ShapeDtypeStruct((NPAD, C), jnp.float32),
            jax.ShapeDtypeStruct((NPAD, C), jnp.float32),
        ],
    )(ivf_p, jnp.pad(xyz_p, ((0, 0), (0, 1))), w1a_t,
      jnp.pad(w1b_t, ((0, 1), (0, 0))), b1_2d)


def _sc_body(a_hbm, q_hbm, ed_hbm,
             m_hbm, st_hbm,
             q_blk, m_blk, st_buf, lst_s, lst_d, arows, stat_st,
             st_sem, g_sem):
    cid = lax.axis_index("c")
    sid = lax.axis_index("s")
    wid = sid * NC + cid
    base = wid * R

    neg = jnp.full((L,), _NEG, jnp.float32)
    zeroi = jnp.zeros((L,), jnp.int32)

    @pl.loop(0, R)
    def _(r):
        for k in range(KC):
            m_blk[r, pl.ds(k * L, L)] = neg

    @pl.loop(0, 2 * LCAP // L)
    def _(i):
        lst_s[pl.ds(i * L, L)] = zeroi
        lst_d[pl.ds(i * L, L)] = zeroi

    pltpu.sync_copy(q_hbm.at[pl.ds(base, R)], q_blk)

    pltpu.make_async_copy(ed_hbm.at[:, pl.ds(0, CH)],
                          st_buf.at[:, pl.ds(0, CH)], st_sem.at[0]).start()
    pltpu.make_async_copy(ed_hbm.at[:, pl.ds(CH, CH)],
                          st_buf.at[:, pl.ds(CH, CH)], st_sem.at[1]).start()

    def chunk_body(c, carry):
        cnt_prev = carry[0]
        stats = carry[1:]
        sl = c % 2
        slp = (c + 1) % 2

        pltpu.make_async_copy(ed_hbm.at[:, pl.ds(0, CH)],
                              st_buf.at[:, pl.ds(sl * CH, CH)],
                              st_sem.at[sl]).wait()

        def fbody(g, pos):
            vs = st_buf[0, pl.ds(sl * CH + g * L, L)]
            vd = st_buf[1, pl.ds(sl * CH + g * L, L)]
            dl = vd - base
            inr = dl.astype(jnp.uint32) < jnp.uint32(R)
            incl = plsc.cumsum(inr.astype(jnp.int32))
            posl = pos + incl - 1
            plsc.store_scatter(lst_s, [sl * LCAP + posl], vs, mask=inr)
            plsc.store_scatter(lst_d, [sl * LCAP + posl], dl, mask=inr)
            return pos + plsc.all_reduce_population_count(inr)

        pos = lax.fori_loop(0, NG, fbody, jnp.zeros((L,), jnp.int32),
                            unroll=2)
        cnt = jnp.max(pos)
        nb = (cnt + (GB - 1)) // GB

        @pl.when(nb > 0)
        def _():
            pltpu.make_async_copy(a_hbm.at[lst_s.at[pl.ds(sl * LCAP, GB)]],
                                  arows.at[pl.ds(sl * GB, GB)],
                                  g_sem.at[sl]).start()

        cnxt = jnp.minimum(c + 2, NCHUNK - 1)
        pltpu.make_async_copy(ed_hbm.at[:, pl.ds(cnxt * CH, CH)],
                              st_buf.at[:, pl.ds(sl * CH, CH)],
                              st_sem.at[sl]).start()

        nbp = (cnt_prev + (GB - 1)) // GB

        def bbody(b, stats):
            @pl.when(b == 0)
            def _():
                pltpu.make_async_copy(a_hbm.at[lst_s.at[pl.ds(slp * LCAP, GB)]],
                                      arows.at[pl.ds(slp * GB, GB)],
                                      g_sem.at[slp]).wait()

            @pl.when(b > 0)
            def _():
                pltpu.sync_copy(a_hbm.at[lst_s.at[pl.ds(slp * LCAP + b * GB, GB)]],
                                arows.at[pl.ds(slp * GB, GB)])

            def gbody(g, stats):
                erow = g * L
                vdv = lst_d[pl.ds(slp * LCAP + b * GB + erow, L)]
                new = list(stats)
                for k in range(L):
                    d = vdv[k]
                    valid = (b * GB + erow + k) < cnt_prev
                    for kc in range(KC):
                        slc = pl.ds(kc * L, L)
                        a = arows[slp * GB + erow + k, slc]
                        a_eff = jnp.where(valid, a, _NEG)
                        m_blk[d, slc] = jnp.maximum(m_blk[d, slc], a_eff)
                        h = jnp.maximum(a_eff - q_blk[d, slc], 0.0)
                        new[kc] = new[kc] + h
                        new[KC + kc] = new[KC + kc] + h * h
                return tuple(new)

            return lax.fori_loop(0, GB // L, gbody, stats)

        stats = lax.fori_loop(0, nbp, bbody, stats)
        return (cnt,) + stats

    carry0 = (jnp.int32(0),) + tuple(
        jnp.zeros((L,), jnp.float32) for _ in range(2 * KC))
    carry = lax.fori_loop(0, NCHUNK + 1, chunk_body, carry0)
    cnt_last = carry[0]
    stats = carry[1:]

    @pl.when(cnt_last > 0)
    def _():
        slx = NCHUNK % 2
        pltpu.make_async_copy(a_hbm.at[lst_s.at[pl.ds(slx * LCAP, GB)]],
                              arows.at[pl.ds(slx * GB, GB)],
                              g_sem.at[slx]).wait()

    pltpu.make_async_copy(ed_hbm.at[:, pl.ds(0, CH)],
                          st_buf.at[:, pl.ds(0, CH)], st_sem.at[0]).wait()
    pltpu.make_async_copy(ed_hbm.at[:, pl.ds(0, CH)],
                          st_buf.at[:, pl.ds(CH, CH)], st_sem.at[1]).wait()

    for k in range(KC):
        slc = pl.ds(k * L, L)
        stat_st[0, slc] = stats[k]
        stat_st[1, slc] = stats[KC + k]

    pltpu.sync_copy(m_blk, m_hbm.at[pl.ds(base, R)])
    pltpu.sync_copy(stat_st, st_hbm.at[wid])


def _sc_call(a, q, ed):
    mesh = plsc.VectorSubcoreMesh(core_axis_name="c", subcore_axis_name="s")
    cp = pltpu.CompilerParams()
    if "needs_layout_passes" in pltpu.CompilerParams.__dataclass_fields__:
        import dataclasses
        cp = dataclasses.replace(cp, needs_layout_passes=False)
    kern = pl.kernel(
        _sc_body,
        out_type=[
            jax.ShapeDtypeStruct((NPAD, C), jnp.float32),
            jax.ShapeDtypeStruct((NW, 2, C), jnp.float32),
        ],
        mesh=mesh,
        scratch_types=[
            pltpu.VMEM((R, C), jnp.float32),
            pltpu.VMEM((R, C), jnp.float32),
            pltpu.VMEM((2, 2 * CH), jnp.int32),
            pltpu.VMEM((2 * LCAP,), jnp.int32),
            pltpu.VMEM((2 * LCAP,), jnp.int32),
            pltpu.VMEM((2 * GB, C), jnp.float32),
            pltpu.VMEM((2, C), jnp.float32),
            pltpu.SemaphoreType.DMA((2,)),
            pltpu.SemaphoreType.DMA((2,)),
        ],
        compiler_params=cp,
    )
    return kern(a, q, ed)


def _tc2_body(m_ref, q_ref, st_ref, ivf_ref, w2t_ref, b2_ref,
              g1_ref, be1_ref, g2_ref, be2_ref, o_ref):
    st = st_ref[...]
    s = jnp.sum(st[:, 0, :], axis=0, keepdims=True)
    ss = jnp.sum(st[:, 1, :], axis=0, keepdims=True)
    mu1 = s / E
    var1 = ss / E - mu1 * mu1
    inv1 = g1_ref[...] * lax.rsqrt(var1 + EPS)

    m = m_ref[...]
    hseg = jnp.maximum(m - q_ref[...], 0.0)
    agg = jnp.where(m == _NEG, 0.0, (hseg - mu1) * inv1 + be1_ref[...])

    u = jnp.dot(agg, w2t_ref[...], preferred_element_type=jnp.float32)
    u = jnp.maximum(u + b2_ref[...], 0.0)

    rows = lax.broadcasted_iota(jnp.int32, (NPAD, 1), 0)
    valid = rows < N
    uv = jnp.where(valid, u, 0.0)
    mu2 = jnp.sum(uv, axis=0, keepdims=True) / N
    dev = jnp.where(valid, u - mu2, 0.0)
    var2 = jnp.sum(dev * dev, axis=0, keepdims=True) / N
    inv2 = g2_ref[...] * lax.rsqrt(var2 + EPS)
    o_ref[...] = (u - mu2) * inv2 + be2_ref[...] + ivf_ref[...]


def _tc2(m, q, st, ivf_p, w2_t, b2_2d, g1_2d, be1_2d, g2_2d, be2_2d):
    return pl.pallas_call(
        _tc2_body,
        out_shape=jax.ShapeDtypeStruct((NPAD, C), jnp.float32),
    )(m, q, st, ivf_p, w2_t, b2_2d, g1_2d, be1_2d, g2_2d, be2_2d)


@jax.jit
def kernel(xyz, features, edges, W1, b1, g1, be1, W2, b2, g2, be2):
    f = jnp.float32
    ivf = features[0].astype(f).T
    ivf_p = jnp.pad(ivf, ((0, NPAD - N), (0, 0)))
    xyz_p = jnp.pad(xyz[0].astype(f), ((0, NPAD - N), (0, 0)))
    ed = edges[0].astype(jnp.int32)

    w1a_t = W1[:, :C].astype(f).T
    w1b_t = W1[:, C:].astype(f).T

    a, q = _tc1(ivf_p, xyz_p, w1a_t, w1b_t, b1.astype(f)[None])
    m, st = _sc_call(a, q, ed)
    res = _tc2(m, q, st, ivf_p, W2.astype(f).T, b2.astype(f)[None],
               g1.astype(f)[None], be1.astype(f)[None],
               g2.astype(f)[None], be2.astype(f)[None])
    return res[:N].T[None]
